# baseline (speedup 1.0000x reference)
import functools

import jax
import jax.numpy as jnp
from jax.experimental import pallas as pl
from jax.experimental.pallas import tpu as pltpu

_ALPHA = 3.0
_K = 64
_ROW_BLOCK = 512


def _nodevec_kernel(scale_ref, e1_ref, e2_ref, w1_ref, b1_ref, w2_ref, b2_ref,
                    nv1_ref, nv2_ref, nv1t_ref, nv2t_ref, *, fnum):
    v1 = e1_ref[...]
    v2 = e2_ref[...]
    for i in range(fnum):
        s = scale_ref[0:1, i:i + 1]
        dn = (((1,), (1,)), ((), ()))
        v1 = jnp.tanh(_ALPHA * (
            jax.lax.dot_general(v1 * s, w1_ref[i], dn,
                                preferred_element_type=jnp.float32)
            + b1_ref[i]))
        v2 = jnp.tanh(_ALPHA * (
            jax.lax.dot_general(v2 * s, w2_ref[i], dn,
                                preferred_element_type=jnp.float32)
            + b2_ref[i]))
        nv1_ref[i] = v1
        nv2_ref[i] = v2
        nv1t_ref[i] = v1.T
        nv2t_ref[i] = v2.T


def _adj_kernel(a1_ref, a2_ref, b1t_ref, b2t_ref, out_ref, *, n, k):
    br = a1_ref.shape[1]
    p = jnp.dot(a1_ref[0], b2t_ref[0], preferred_element_type=jnp.float32)
    q = jnp.dot(a2_ref[0], b1t_ref[0], preferred_element_type=jnp.float32)
    a = p - q
    v = jnp.where(a > 0, jnp.tanh(_ALPHA * a), 0.0)
    bits = jax.lax.bitcast_convert_type(v, jnp.int32)

    macc = v[:, 0:128]
    for c in range(1, n // 128):
        macc = jnp.maximum(macc, v[:, c * 128:(c + 1) * 128])
    macc_pair = jnp.maximum(macc, pltpu.roll(macc, 1, 1))
    pair_sel = (jax.lax.broadcasted_iota(jnp.int32, macc.shape, 1) % 2) == 1
    rmin = jnp.min(jnp.where(pair_sel, macc_pair, 1.0), axis=1, keepdims=True)
    rmax = jnp.max(macc, axis=1, keepdims=True)
    lo0 = jax.lax.bitcast_convert_type(rmin, jnp.int32)
    hi0 = jax.lax.bitcast_convert_type(rmax, jnp.int32)

    def count_ge(x):
        h = n // 2
        ca = jnp.sum((bits[:, :h] >= x).astype(jnp.int32), axis=1,
                     keepdims=True)
        cb = jnp.sum((bits[:, h:] >= x).astype(jnp.int32), axis=1,
                     keepdims=True)
        return ca + cb

    clo0 = count_ge(lo0)

    def cond(carry):
        lo, hi, clo = carry
        return jnp.any(lo < hi)

    def body(carry):
        lo, hi, clo = carry
        mid = lo + ((hi - lo + 1) >> 1)
        cnt = count_ge(mid)
        upd = lo < hi
        ge = cnt >= k
        lo = jnp.where(upd & ge, mid, lo)
        clo = jnp.where(upd & ge, cnt, clo)
        hi = jnp.where(upd & ~ge, mid - 1, hi)
        return lo, hi, clo

    t, _, ct = jax.lax.while_loop(cond, body, (lo0, hi0, clo0))

    def no_ties():
        out_ref[0] = jnp.where(bits >= t, v, 0.0)

    def with_ties():
        gt = bits > t
        cnt_gt = jnp.sum(gt.astype(jnp.int32), axis=1, keepdims=True)
        m = (k - cnt_gt).astype(jnp.float32)
        eq = bits == t

        ri = jax.lax.broadcasted_iota(jnp.int32, (128, 128), 0)
        ci = jax.lax.broadcasted_iota(jnp.int32, (128, 128), 1)
        u128 = (ri < ci).astype(jnp.bfloat16)
        running = jnp.zeros((br, 1), jnp.float32)
        for c in range(n // 128):
            sl = slice(c * 128, (c + 1) * 128)
            eq_c = eq[:, sl]
            eqf_c = eq_c.astype(jnp.bfloat16)
            within_c = jnp.dot(eqf_c, u128,
                               preferred_element_type=jnp.float32)
            rank_c = within_c + running
            keep_c = gt[:, sl] | (eq_c & (rank_c < m))
            out_ref[0, :, sl] = jnp.where(keep_c, v[:, sl], 0.0)
            running = rank_c[:, 127:128] + eqf_c[:, 127:128].astype(
                jnp.float32)

    jax.lax.cond(jnp.all(ct == k), no_ties, with_ties)


@functools.partial(jax.jit, static_argnames=())
def kernel(idx, scale_set, emb1, emb2, W1, b1, W2, b2):
    n, dim = emb1.shape
    fnum = W1.shape[0]
    br = min(_ROW_BLOCK, n)
    nb = n // br

    nv1 = jnp.take(emb1, idx, axis=0)
    nv2 = jnp.take(emb2, idx, axis=0)

    nv1o, nv2o, nv1t, nv2t = pl.pallas_call(
        functools.partial(_nodevec_kernel, fnum=fnum),
        grid=(nb,),
        in_specs=[
            pl.BlockSpec((1, fnum), lambda r: (0, 0)),
            pl.BlockSpec((br, dim), lambda r: (r, 0)),
            pl.BlockSpec((br, dim), lambda r: (r, 0)),
            pl.BlockSpec((fnum, dim, dim), lambda r: (0, 0, 0)),
            pl.BlockSpec((fnum, 1, dim), lambda r: (0, 0, 0)),
            pl.BlockSpec((fnum, dim, dim), lambda r: (0, 0, 0)),
            pl.BlockSpec((fnum, 1, dim), lambda r: (0, 0, 0)),
        ],
        out_specs=[
            pl.BlockSpec((fnum, br, dim), lambda r: (0, r, 0)),
            pl.BlockSpec((fnum, br, dim), lambda r: (0, r, 0)),
            pl.BlockSpec((fnum, dim, br), lambda r: (0, 0, r)),
            pl.BlockSpec((fnum, dim, br), lambda r: (0, 0, r)),
        ],
        out_shape=[
            jax.ShapeDtypeStruct((fnum, n, dim), jnp.float32),
            jax.ShapeDtypeStruct((fnum, n, dim), jnp.float32),
            jax.ShapeDtypeStruct((fnum, dim, n), jnp.float32),
            jax.ShapeDtypeStruct((fnum, dim, n), jnp.float32),
        ],
    )(scale_set.reshape(1, fnum), nv1, nv2, W1, b1.reshape(fnum, 1, dim),
      W2, b2.reshape(fnum, 1, dim))

    adj = pl.pallas_call(
        functools.partial(_adj_kernel, n=n, k=_K),
        grid=(fnum, nb),
        in_specs=[
            pl.BlockSpec((1, br, dim), lambda i, r: (i, r, 0)),
            pl.BlockSpec((1, br, dim), lambda i, r: (i, r, 0)),
            pl.BlockSpec((1, dim, n), lambda i, r: (i, 0, 0)),
            pl.BlockSpec((1, dim, n), lambda i, r: (i, 0, 0)),
        ],
        out_specs=pl.BlockSpec((1, br, n), lambda i, r: (i, r, 0)),
        out_shape=jax.ShapeDtypeStruct((fnum, n, n), jnp.float32),
    )(nv1o, nv2o, nv1t, nv2t)

    return tuple(adj[i] for i in range(fnum))


# speedup vs baseline: 1.0224x; 1.0224x over previous
import functools

import jax
import jax.numpy as jnp
from jax.experimental import pallas as pl
from jax.experimental.pallas import tpu as pltpu

_ALPHA = 3.0
_K = 64
_ROW_BLOCK = 512


def _nodevec_kernel(scale_ref, e1_ref, e2_ref, w1_ref, b1_ref, w2_ref, b2_ref,
                    nv1_ref, nv2_ref, nv1t_ref, nv2t_ref, *, fnum):
    v1 = e1_ref[...]
    v2 = e2_ref[...]
    for i in range(fnum):
        s = scale_ref[0:1, i:i + 1]
        dn = (((1,), (1,)), ((), ()))
        v1 = jnp.tanh(_ALPHA * (
            jax.lax.dot_general(v1 * s, w1_ref[i], dn,
                                preferred_element_type=jnp.float32)
            + b1_ref[i]))
        v2 = jnp.tanh(_ALPHA * (
            jax.lax.dot_general(v2 * s, w2_ref[i], dn,
                                preferred_element_type=jnp.float32)
            + b2_ref[i]))
        nv1_ref[i] = v1
        nv2_ref[i] = v2
        nv1t_ref[i] = v1.T
        nv2t_ref[i] = v2.T


def _adj_kernel(a1_ref, a2_ref, b1t_ref, b2t_ref, out_ref, *, n, k):
    br = a1_ref.shape[1]
    p = jnp.dot(a1_ref[0], b2t_ref[0], preferred_element_type=jnp.float32)
    q = jnp.dot(a2_ref[0], b1t_ref[0], preferred_element_type=jnp.float32)
    a = p - q
    v = jnp.where(a > 0, jnp.tanh(_ALPHA * a), 0.0)
    bits = jax.lax.bitcast_convert_type(v, jnp.int32)

    macc = v[:, 0:128]
    for c in range(1, n // 128):
        macc = jnp.maximum(macc, v[:, c * 128:(c + 1) * 128])
    macc_pair = jnp.maximum(macc, pltpu.roll(macc, 1, 1))
    pair_sel = (jax.lax.broadcasted_iota(jnp.int32, macc.shape, 1) % 2) == 1
    rmin = jnp.min(jnp.where(pair_sel, macc_pair, 1.0), axis=1, keepdims=True)
    rmax = jnp.max(macc, axis=1, keepdims=True)
    lo0 = jax.lax.bitcast_convert_type(rmin, jnp.int32)
    hi0 = jax.lax.bitcast_convert_type(rmax, jnp.int32)

    def count_ge(x):
        return jnp.sum((bits >= x).astype(jnp.int32), axis=1, keepdims=True)

    clo0 = count_ge(lo0)

    def cond(carry):
        lo, hi, clo = carry
        return jnp.any(lo < hi)

    def body(carry):
        lo, hi, clo = carry
        mid = lo + ((hi - lo + 1) >> 1)
        cnt = count_ge(mid)
        upd = lo < hi
        ge = cnt >= k
        lo = jnp.where(upd & ge, mid, lo)
        clo = jnp.where(upd & ge, cnt, clo)
        hi = jnp.where(upd & ~ge, mid - 1, hi)
        return lo, hi, clo

    t, _, ct = jax.lax.while_loop(cond, body, (lo0, hi0, clo0))

    def no_ties():
        out_ref[0] = jnp.where(bits >= t, v, 0.0)

    def with_ties():
        gt = bits > t
        cnt_gt = jnp.sum(gt.astype(jnp.int32), axis=1, keepdims=True)
        m = (k - cnt_gt).astype(jnp.float32)
        eq = bits == t

        ri = jax.lax.broadcasted_iota(jnp.int32, (128, 128), 0)
        ci = jax.lax.broadcasted_iota(jnp.int32, (128, 128), 1)
        u128 = (ri < ci).astype(jnp.bfloat16)
        running = jnp.zeros((br, 1), jnp.float32)
        for c in range(n // 128):
            sl = slice(c * 128, (c + 1) * 128)
            eq_c = eq[:, sl]
            eqf_c = eq_c.astype(jnp.bfloat16)
            within_c = jnp.dot(eqf_c, u128,
                               preferred_element_type=jnp.float32)
            rank_c = within_c + running
            keep_c = gt[:, sl] | (eq_c & (rank_c < m))
            out_ref[0, :, sl] = jnp.where(keep_c, v[:, sl], 0.0)
            running = rank_c[:, 127:128] + eqf_c[:, 127:128].astype(
                jnp.float32)

    jax.lax.cond(jnp.all(ct == k), no_ties, with_ties)


@functools.partial(jax.jit, static_argnames=())
def kernel(idx, scale_set, emb1, emb2, W1, b1, W2, b2):
    n, dim = emb1.shape
    fnum = W1.shape[0]
    br = min(_ROW_BLOCK, n)
    nb = n // br

    nv1 = jnp.take(emb1, idx, axis=0)
    nv2 = jnp.take(emb2, idx, axis=0)

    nv1o, nv2o, nv1t, nv2t = pl.pallas_call(
        functools.partial(_nodevec_kernel, fnum=fnum),
        grid=(nb,),
        in_specs=[
            pl.BlockSpec((1, fnum), lambda r: (0, 0)),
            pl.BlockSpec((br, dim), lambda r: (r, 0)),
            pl.BlockSpec((br, dim), lambda r: (r, 0)),
            pl.BlockSpec((fnum, dim, dim), lambda r: (0, 0, 0)),
            pl.BlockSpec((fnum, 1, dim), lambda r: (0, 0, 0)),
            pl.BlockSpec((fnum, dim, dim), lambda r: (0, 0, 0)),
            pl.BlockSpec((fnum, 1, dim), lambda r: (0, 0, 0)),
        ],
        out_specs=[
            pl.BlockSpec((fnum, br, dim), lambda r: (0, r, 0)),
            pl.BlockSpec((fnum, br, dim), lambda r: (0, r, 0)),
            pl.BlockSpec((fnum, dim, br), lambda r: (0, 0, r)),
            pl.BlockSpec((fnum, dim, br), lambda r: (0, 0, r)),
        ],
        out_shape=[
            jax.ShapeDtypeStruct((fnum, n, dim), jnp.float32),
            jax.ShapeDtypeStruct((fnum, n, dim), jnp.float32),
            jax.ShapeDtypeStruct((fnum, dim, n), jnp.float32),
            jax.ShapeDtypeStruct((fnum, dim, n), jnp.float32),
        ],
    )(scale_set.reshape(1, fnum), nv1, nv2, W1, b1.reshape(fnum, 1, dim),
      W2, b2.reshape(fnum, 1, dim))

    adj = pl.pallas_call(
        functools.partial(_adj_kernel, n=n, k=_K),
        grid=(fnum, nb),
        in_specs=[
            pl.BlockSpec((1, br, dim), lambda i, r: (i, r, 0)),
            pl.BlockSpec((1, br, dim), lambda i, r: (i, r, 0)),
            pl.BlockSpec((1, dim, n), lambda i, r: (i, 0, 0)),
            pl.BlockSpec((1, dim, n), lambda i, r: (i, 0, 0)),
        ],
        out_specs=pl.BlockSpec((1, br, n), lambda i, r: (i, r, 0)),
        out_shape=jax.ShapeDtypeStruct((fnum, n, n), jnp.float32),
    )(nv1o, nv2o, nv1t, nv2t)

    return tuple(adj[i] for i in range(fnum))
